# Initial kernel scaffold; baseline (speedup 1.0000x reference)
import jax
import jax.numpy as jnp
from jax.experimental import pallas as pl


def kernel(t_in, y_in, alpha_1st, beta_1st, gamma_1st, alpha_2nd, beta_2nd, gamma_2nd, w_T, b_T, w_d, b_d, inds_r1_1st, inds_p1_1st, inds_p2_1st, inds_r1_2nd, inds_r2_2nd, inds_p1_2nd, inds_p2_2nd):
    raise NotImplementedError("write your pallas kernel here")


# speedup vs baseline: 5.6260x; 5.6260x over previous
import functools

import jax
import jax.numpy as jnp
from jax import lax
from jax.experimental import pallas as pl
from jax.experimental.pallas import tpu as pltpu

_RB = 512


def _body(t_ref, wT_ref, bT_ref, wd_ref, bd_ref, y_ref,
          al_ref, be_ref, ga_ref, r1_ref, r2_ref, p1_ref, p2_ref,
          out_ref, *, n_spec):
    i = pl.program_id(0)

    t = t_ref[...]
    T = jnp.exp(wT_ref[0] * t + bT_ref[0]) + 10.0
    den = jnp.exp(wd_ref[0] * t + bd_ref[0])
    logT = jnp.log(T / 300.0)
    nTinv = -1.0 / T

    al = al_ref[0]
    be = be_ref[0]
    ga = ga_ref[0]
    rates = al * jnp.exp(be * logT + ga * nTinv)

    r1 = r1_ref[0]
    r2 = r2_ref[0]
    p1 = p1_ref[0]
    p2 = p2_ref[0]

    spec = lax.broadcasted_iota(jnp.int32, (n_spec, _RB), 0)
    G1 = (spec == r1).astype(jnp.float32)
    G2 = (spec == r2).astype(jnp.float32)
    P1 = (spec == p1).astype(jnp.float32)
    P2 = (spec == p2).astype(jnp.float32)

    y = y_ref[...]
    Y1 = jnp.dot(y, G1, preferred_element_type=jnp.float32)
    Y2 = jnp.dot(y, G2, preferred_element_type=jnp.float32)

    is1st = (r2 == n_spec).astype(jnp.float32)
    factor = den * Y2 + is1st
    term = rates * Y1 * factor

    S = P1 + P2 - G1 - G2
    contrib = lax.dot_general(term, S, (((1,), (1,)), ((), ())),
                              preferred_element_type=jnp.float32)

    @pl.when(i == 0)
    def _():
        out_ref[...] = jnp.zeros_like(out_ref)

    out_ref[...] += contrib


def kernel(t_in, y_in, alpha_1st, beta_1st, gamma_1st, alpha_2nd, beta_2nd,
           gamma_2nd, w_T, b_T, w_d, b_d, inds_r1_1st, inds_p1_1st,
           inds_p2_1st, inds_r1_2nd, inds_r2_2nd, inds_p1_2nd, inds_p2_2nd):
    B = t_in.shape[0]
    n_spec = y_in.shape[1]
    Rl = alpha_1st.shape[0]
    i32 = jnp.int32

    al = jnp.concatenate([alpha_1st, alpha_2nd])
    be = jnp.concatenate([beta_1st, beta_2nd])
    ga = jnp.concatenate([gamma_1st, gamma_2nd])
    r1 = jnp.concatenate([inds_r1_1st.astype(i32), inds_r1_2nd.astype(i32)])
    r2 = jnp.concatenate([jnp.full((Rl,), n_spec, i32), inds_r2_2nd.astype(i32)])
    p1 = jnp.concatenate([inds_p1_1st.astype(i32), inds_p1_2nd.astype(i32)])
    p2 = jnp.concatenate([inds_p2_1st.astype(i32), inds_p2_2nd.astype(i32)])

    R = al.shape[0]
    nb = -(-R // _RB)
    pad = nb * _RB - R

    def padded(x, fill):
        x = jnp.pad(x, (0, pad), constant_values=fill)
        return x.reshape(nb, 1, _RB)

    al, be, ga = padded(al, 0.0), padded(be, 0.0), padded(ga, 0.0)
    r1, r2, p1, p2 = (padded(v, 0) for v in (r1, r2, p1, p2))

    t2 = t_in.reshape(B, 1)

    full2d = lambda shape: pl.BlockSpec(shape, lambda i: (0, 0))
    par3d = pl.BlockSpec((1, 1, _RB), lambda i: (i, 0, 0))
    smem = pl.BlockSpec(memory_space=pltpu.SMEM)

    out = pl.pallas_call(
        functools.partial(_body, n_spec=n_spec),
        grid=(nb,),
        in_specs=[full2d((B, 1)), smem, smem, smem, smem, full2d((B, n_spec)),
                  par3d, par3d, par3d, par3d, par3d, par3d, par3d],
        out_specs=pl.BlockSpec((B, n_spec), lambda i: (0, 0)),
        out_shape=jax.ShapeDtypeStruct((B, n_spec), jnp.float32),
    )(t2, w_T, b_T, w_d, b_d, y_in, al, be, ga, r1, r2, p1, p2)
    return out
